# baseline (speedup 1.0000x reference)
import functools

import jax
import jax.numpy as jnp
from jax import lax
from jax.experimental import pallas as pl
from jax.experimental.pallas import tpu as pltpu
from jax.experimental.pallas import tpu_sc as plsc


def _sc_copy(w_hbm, o_hbm, sems, *, rows_w, chunk, n_chunks):
    wid = lax.axis_index("s") * 2 + lax.axis_index("c")
    base = wid * rows_w

    def copy(c):
        return pltpu.make_async_copy(
            w_hbm.at[pl.ds(base + c * chunk, chunk), :],
            o_hbm.at[pl.ds(base + c * chunk, chunk), :],
            sems.at[c],
        )

    for c in range(n_chunks):
        copy(c).start()
    for c in range(n_chunks):
        copy(c).wait()


def kernel(x, W):
    del x
    S, D = W.shape
    n_workers = 32
    rows_w = S // n_workers
    chunk = 64
    n_chunks = rows_w // chunk
    mesh = plsc.VectorSubcoreMesh(core_axis_name="c", subcore_axis_name="s")
    body = functools.partial(
        _sc_copy, rows_w=rows_w, chunk=chunk, n_chunks=n_chunks)
    k = pl.kernel(
        body,
        out_type=jax.ShapeDtypeStruct((S, D), W.dtype),
        mesh=mesh,
        scratch_types=[
            pltpu.SemaphoreType.DMA((n_chunks,)),
        ],
    )
    return k(W)


# speedup vs baseline: 49.3156x; 49.3156x over previous
import jax
import jax.numpy as jnp
from jax.experimental import pallas as pl
from jax.experimental.pallas import tpu as pltpu


def _copy_block(w_ref, o_ref):
    o_ref[...] = w_ref[...]


def kernel(x, W):
    del x
    S, D = W.shape
    blk = 4096
    return pl.pallas_call(
        _copy_block,
        grid=(S // blk,),
        in_specs=[pl.BlockSpec((blk, D), lambda i: (i, 0))],
        out_specs=pl.BlockSpec((blk, D), lambda i: (i, 0)),
        out_shape=jax.ShapeDtypeStruct((S, D), W.dtype),
        compiler_params=pltpu.CompilerParams(
            dimension_semantics=("parallel",),
        ),
    )(W)
